# baseline (speedup 1.0000x reference)
import jax
import jax.numpy as jnp
from jax import lax
from jax.experimental import pallas as pl
from jax.experimental.pallas import tpu as pltpu
from jax.experimental.pallas import tpu_sc as plsc

D = 128
DP = 144
NC = 2
NS = 16
L = 16
SUB = 128


def _sc_body_maker(n_pad, units_c0, units_c1):
    stripe = n_pad // NS

    def body(feat_hbm, edges_hbm, zeros_hbm, pacc_hbm,
             idx_v, rows0, rows1, acc_sh,
             gsem0, gsem1, isem0, isem1, isem2, isem3):
        c = lax.axis_index("c")
        s = lax.axis_index("s")
        rows = (rows0, rows1)
        gsems = (gsem0, gsem1)
        isems = (isem0, isem1, isem2, isem3)

        pltpu.sync_copy(zeros_hbm, rows0.at[pl.ds(0, L)])

        @pl.loop(0, stripe // L)
        def _zero(k):
            pltpu.sync_copy(rows0.at[pl.ds(0, L)],
                            acc_sh.at[pl.ds(s * stripe + k * L, L)])

        plsc.subcore_barrier()

        def pipeline(units, row_base):
            g_desc = {}
            i_desc = {}

            def idx_start(k):
                i_desc[k] = pltpu.async_copy(
                    edges_hbm.at[row_base + k], idx_v.at[k % 4],
                    isems[k % 4])

            def gather_start(k):
                g_desc[k] = pltpu.async_copy(
                    feat_hbm.at[idx_v.at[k % 4, 0]], rows[k % 2],
                    gsems[k % 2])

            def scatter_sync(k):
                pltpu.sync_copy(rows[k % 2],
                                acc_sh.at[idx_v.at[k % 4, 1]], add=True)

            idx_start(0)
            i_desc[0].wait()
            idx_start(1)
            idx_start(2)
            gather_start(0)
            for k in range(units):
                g_desc[k].wait()
                if k + 3 < units:
                    idx_start(k + 3)
                if k + 1 < units:
                    i_desc[k + 1].wait()
                    gather_start(k + 1)
                scatter_sync(k)

        @pl.when(c == 0)
        def _c0():
            pipeline(units_c0, s * units_c0)

        @pl.when(c == 1)
        def _c1():
            pipeline(units_c1, NS * units_c0 + s * units_c1)

        plsc.subcore_barrier()
        pltpu.sync_copy(acc_sh.at[pl.ds(s * stripe, stripe)],
                        pacc_hbm.at[c, pl.ds(s * stripe, stripe)])

    return body


def _tc_body(feat_ref, pacc_ref, ws_ref, wn_ref, bs_ref, bn_ref,
             out_ref):
    x = feat_ref[...]
    a = pacc_ref[0] + pacc_ref[1]
    ssum = a[:, :D]
    cnt = a[:, D:D + 1]
    h = ssum / jnp.maximum(cnt, 1.0)
    out_ref[...] = (
        jnp.dot(x, ws_ref[...], preferred_element_type=jnp.float32)
        + jnp.dot(h, wn_ref[...], preferred_element_type=jnp.float32)
        + bs_ref[...] + bn_ref[...]
    )


def kernel(feat, edge_index, W_self, b_self, W_neigh, b_neigh):
    n, d = feat.shape
    e = edge_index.shape[1]
    assert d == D

    n_pad = -(-n // (NS * L)) * (NS * L)
    tile_e = -(-e // (NC * NS * SUB)) * SUB
    e_pad = tile_e * NC * NS
    units = tile_e // SUB
    units_c0 = 36
    units_c1 = 2 * units - units_c0
    blk = 400
    n_blocks = n // blk

    src = edge_index[0]
    dst = edge_index[1]
    pad_e = e_pad - e
    src_p = jnp.concatenate(
        [src, jnp.zeros((pad_e,), jnp.int32)]).reshape(e_pad // SUB, SUB)
    dst_p = jnp.concatenate(
        [dst, jnp.full((pad_e,), n_pad - 1, jnp.int32)]
    ).reshape(e_pad // SUB, SUB)
    edges3 = jnp.stack([src_p, dst_p], axis=1)
    featx = jnp.pad(feat, ((0, 0), (0, DP - D)), constant_values=1.0)
    zeros_blk = jnp.zeros((L, DP), jnp.float32)

    mesh = plsc.VectorSubcoreMesh(core_axis_name="c", subcore_axis_name="s")
    sc_call = pl.kernel(
        _sc_body_maker(n_pad, units_c0, units_c1),
        out_type=jax.ShapeDtypeStruct((NC, n_pad, DP), jnp.float32),
        mesh=mesh,
        scratch_types=[
            pltpu.VMEM((4, 2, SUB), jnp.int32),
            pltpu.VMEM((SUB, DP), jnp.float32),
            pltpu.VMEM((SUB, DP), jnp.float32),
            pltpu.VMEM_SHARED((n_pad, DP), jnp.float32),
        ] + [pltpu.SemaphoreType.DMA] * 6,
        compiler_params=pltpu.CompilerParams(use_tc_tiling_on_sc=False),
        name="sage_sc_aggregate",
    )
    pacc = sc_call(featx, edges3, zeros_blk)

    out = pl.pallas_call(
        _tc_body,
        grid=(n_blocks,),
        in_specs=[
            pl.BlockSpec((blk, D), lambda i: (i, 0)),
            pl.BlockSpec((NC, blk, DP), lambda i: (0, i, 0)),
            pl.BlockSpec((D, D), lambda i: (0, 0)),
            pl.BlockSpec((D, D), lambda i: (0, 0)),
            pl.BlockSpec((1, D), lambda i: (0, 0)),
            pl.BlockSpec((1, D), lambda i: (0, 0)),
        ],
        out_specs=pl.BlockSpec((blk, D), lambda i: (i, 0)),
        out_shape=jax.ShapeDtypeStruct((n, D), jnp.float32),
        name="sage_tc_combine",
    )(feat, pacc, W_self.T, W_neigh.T,
      b_self.reshape(1, D), b_neigh.reshape(1, D))
    return out


# speedup vs baseline: 1.2161x; 1.2161x over previous
import jax
import jax.numpy as jnp
from jax import lax
from jax.experimental import pallas as pl
from jax.experimental.pallas import tpu as pltpu
from jax.experimental.pallas import tpu_sc as plsc

D = 128
DP = 144
NC = 2
NS = 16
L = 16
SUB = 128


def _sc_body_maker(n_pad, units_c0, units_c1):
    stripe = n_pad // NS

    def body(feat_hbm, edges_hbm, zeros_hbm, pacc_hbm,
             idx_v, rows0, rows1, acc_sh,
             gsem0, gsem1, isem0, isem1, isem2, isem3):
        c = lax.axis_index("c")
        s = lax.axis_index("s")
        rows = (rows0, rows1)
        gsems = (gsem0, gsem1)
        isems = (isem0, isem1, isem2, isem3)

        pltpu.sync_copy(zeros_hbm, rows0.at[pl.ds(0, L)])

        @pl.loop(0, stripe // L)
        def _zero(k):
            pltpu.sync_copy(rows0.at[pl.ds(0, L)],
                            acc_sh.at[pl.ds(s * stripe + k * L, L)])

        plsc.subcore_barrier()

        def pipeline(units, row_base):
            g_desc = {}
            i_desc = {}

            def idx_start(k):
                i_desc[k] = pltpu.async_copy(
                    edges_hbm.at[row_base + k], idx_v.at[k % 4],
                    isems[k % 4])

            def gather_start(k):
                g_desc[k] = pltpu.async_copy(
                    feat_hbm.at[idx_v.at[k % 4, 0]], rows[k % 2],
                    gsems[k % 2])

            def scatter_sync(k):
                pltpu.sync_copy(rows[k % 2],
                                acc_sh.at[idx_v.at[k % 4, 1]], add=True)

            idx_start(0)
            i_desc[0].wait()
            idx_start(1)
            idx_start(2)
            gather_start(0)
            for k in range(units):
                g_desc[k].wait()
                if k + 3 < units:
                    idx_start(k + 3)
                if k + 1 < units:
                    i_desc[k + 1].wait()
                    gather_start(k + 1)
                scatter_sync(k)

        @pl.when(c == 0)
        def _c0():
            pipeline(units_c0, s * units_c0)

        @pl.when(c == 1)
        def _c1():
            pipeline(units_c1, NS * units_c0 + s * units_c1)

        plsc.subcore_barrier()
        pltpu.sync_copy(acc_sh.at[pl.ds(s * stripe, stripe)],
                        pacc_hbm.at[c, pl.ds(s * stripe, stripe)])

    return body


def _tc_body(feat_ref, pacc_ref, ws_ref, wn_ref, bs_ref, bn_ref,
             out_ref):
    x = feat_ref[...]
    a = pacc_ref[0] + pacc_ref[1]
    ssum = a[:, :D]
    cnt = a[:, D:D + 1]
    h = ssum / jnp.maximum(cnt, 1.0)
    out_ref[...] = (
        jnp.dot(x, ws_ref[...], preferred_element_type=jnp.float32)
        + jnp.dot(h, wn_ref[...], preferred_element_type=jnp.float32)
        + bs_ref[...] + bn_ref[...]
    )


def kernel(feat, edge_index, W_self, b_self, W_neigh, b_neigh):
    n, d = feat.shape
    e = edge_index.shape[1]
    assert d == D

    n_pad = -(-n // (NS * L)) * (NS * L)
    tile_e = -(-e // (NC * NS * SUB)) * SUB
    e_pad = tile_e * NC * NS
    units = tile_e // SUB
    units_c0 = 112
    units_c1 = 2 * units - units_c0
    blk = 400
    n_blocks = n // blk

    src = edge_index[0]
    dst = edge_index[1]
    pad_e = e_pad - e
    src_p = jnp.concatenate(
        [src, jnp.zeros((pad_e,), jnp.int32)]).reshape(e_pad // SUB, SUB)
    dst_p = jnp.concatenate(
        [dst, jnp.full((pad_e,), n_pad - 1, jnp.int32)]
    ).reshape(e_pad // SUB, SUB)
    edges3 = jnp.stack([src_p, dst_p], axis=1)
    featx = jnp.pad(feat, ((0, 0), (0, DP - D)), constant_values=1.0)
    zeros_blk = jnp.zeros((L, DP), jnp.float32)

    mesh = plsc.VectorSubcoreMesh(core_axis_name="c", subcore_axis_name="s")
    sc_call = pl.kernel(
        _sc_body_maker(n_pad, units_c0, units_c1),
        out_type=jax.ShapeDtypeStruct((NC, n_pad, DP), jnp.float32),
        mesh=mesh,
        scratch_types=[
            pltpu.VMEM((4, 2, SUB), jnp.int32),
            pltpu.VMEM((SUB, DP), jnp.float32),
            pltpu.VMEM((SUB, DP), jnp.float32),
            pltpu.VMEM_SHARED((n_pad, DP), jnp.float32),
        ] + [pltpu.SemaphoreType.DMA] * 6,
        compiler_params=pltpu.CompilerParams(use_tc_tiling_on_sc=False),
        name="sage_sc_aggregate",
    )
    pacc = sc_call(featx, edges3, zeros_blk)

    out = pl.pallas_call(
        _tc_body,
        grid=(n_blocks,),
        in_specs=[
            pl.BlockSpec((blk, D), lambda i: (i, 0)),
            pl.BlockSpec((NC, blk, DP), lambda i: (0, i, 0)),
            pl.BlockSpec((D, D), lambda i: (0, 0)),
            pl.BlockSpec((D, D), lambda i: (0, 0)),
            pl.BlockSpec((1, D), lambda i: (0, 0)),
            pl.BlockSpec((1, D), lambda i: (0, 0)),
        ],
        out_specs=pl.BlockSpec((blk, D), lambda i: (i, 0)),
        out_shape=jax.ShapeDtypeStruct((n, D), jnp.float32),
        name="sage_tc_combine",
    )(feat, pacc, W_self.T, W_neigh.T,
      b_self.reshape(1, D), b_neigh.reshape(1, D))
    return out


# speedup vs baseline: 1.2233x; 1.0059x over previous
import jax
import jax.numpy as jnp
from jax import lax
from jax.experimental import pallas as pl
from jax.experimental.pallas import tpu as pltpu
from jax.experimental.pallas import tpu_sc as plsc

D = 128
DP = 144
NC = 2
NS = 16
L = 16
SUB = 128


def _sc_body_maker(n_pad, units_c0, units_c1):
    stripe = n_pad // NS

    def body(feat_hbm, edges_hbm, zeros_hbm, pacc_hbm,
             idx_v, rows0, rows1, acc_sh,
             gsem0, gsem1, isem0, isem1, isem2, isem3):
        c = lax.axis_index("c")
        s = lax.axis_index("s")
        rows = (rows0, rows1)
        gsems = (gsem0, gsem1)
        isems = (isem0, isem1, isem2, isem3)

        pltpu.sync_copy(zeros_hbm, rows0.at[pl.ds(0, L)])
        zcps = [
            pltpu.async_copy(rows0.at[pl.ds(0, L)],
                             acc_sh.at[pl.ds(s * stripe + k * L, L)],
                             gsem0)
            for k in range(stripe // L)
        ]
        for cp in zcps:
            cp.wait()

        plsc.subcore_barrier()

        def pipeline(units, row_base):
            g_desc = {}
            i_desc = {}

            def idx_start(k):
                i_desc[k] = pltpu.async_copy(
                    edges_hbm.at[row_base + k], idx_v.at[k % 4],
                    isems[k % 4])

            def gather_start(k):
                g_desc[k] = pltpu.async_copy(
                    feat_hbm.at[idx_v.at[k % 4, 0]], rows[k % 2],
                    gsems[k % 2])

            def scatter_sync(k):
                pltpu.sync_copy(rows[k % 2],
                                acc_sh.at[idx_v.at[k % 4, 1]], add=True)

            idx_start(0)
            i_desc[0].wait()
            idx_start(1)
            idx_start(2)
            gather_start(0)
            for k in range(units):
                g_desc[k].wait()
                if k + 3 < units:
                    idx_start(k + 3)
                if k + 1 < units:
                    i_desc[k + 1].wait()
                    gather_start(k + 1)
                scatter_sync(k)

        @pl.when(c == 0)
        def _c0():
            pipeline(units_c0, s * units_c0)

        @pl.when(c == 1)
        def _c1():
            pipeline(units_c1, NS * units_c0 + s * units_c1)

        plsc.subcore_barrier()
        pltpu.sync_copy(acc_sh.at[pl.ds(s * stripe, stripe)],
                        pacc_hbm.at[c, pl.ds(s * stripe, stripe)])

    return body


def _tc_body(feat_ref, pacc_ref, ws_ref, wn_ref, bs_ref, bn_ref,
             out_ref):
    x = feat_ref[...]
    a = pacc_ref[0] + pacc_ref[1]
    ssum = a[:, :D]
    cnt = a[:, D:D + 1]
    h = ssum / jnp.maximum(cnt, 1.0)
    out_ref[...] = (
        jnp.dot(x, ws_ref[...], preferred_element_type=jnp.float32)
        + jnp.dot(h, wn_ref[...], preferred_element_type=jnp.float32)
        + bs_ref[...] + bn_ref[...]
    )


def kernel(feat, edge_index, W_self, b_self, W_neigh, b_neigh):
    n, d = feat.shape
    e = edge_index.shape[1]
    assert d == D

    n_pad = -(-n // (NS * L)) * (NS * L)
    tile_e = -(-e // (NC * NS * SUB)) * SUB
    e_pad = tile_e * NC * NS
    units = tile_e // SUB
    units_c0 = 112
    units_c1 = 2 * units - units_c0
    blk = 400
    n_blocks = n // blk

    src = edge_index[0]
    dst = edge_index[1]
    pad_e = e_pad - e
    src_p = jnp.concatenate(
        [src, jnp.zeros((pad_e,), jnp.int32)]).reshape(e_pad // SUB, SUB)
    dst_p = jnp.concatenate(
        [dst, jnp.full((pad_e,), n_pad - 1, jnp.int32)]
    ).reshape(e_pad // SUB, SUB)
    edges3 = jnp.stack([src_p, dst_p], axis=1)
    featx = jnp.pad(feat, ((0, 0), (0, DP - D)), constant_values=1.0)
    zeros_blk = jnp.zeros((L, DP), jnp.float32)

    mesh = plsc.VectorSubcoreMesh(core_axis_name="c", subcore_axis_name="s")
    sc_call = pl.kernel(
        _sc_body_maker(n_pad, units_c0, units_c1),
        out_type=jax.ShapeDtypeStruct((NC, n_pad, DP), jnp.float32),
        mesh=mesh,
        scratch_types=[
            pltpu.VMEM((4, 2, SUB), jnp.int32),
            pltpu.VMEM((SUB, DP), jnp.float32),
            pltpu.VMEM((SUB, DP), jnp.float32),
            pltpu.VMEM_SHARED((n_pad, DP), jnp.float32),
        ] + [pltpu.SemaphoreType.DMA] * 6,
        compiler_params=pltpu.CompilerParams(use_tc_tiling_on_sc=False),
        name="sage_sc_aggregate",
    )
    pacc = sc_call(featx, edges3, zeros_blk)

    out = pl.pallas_call(
        _tc_body,
        grid=(n_blocks,),
        in_specs=[
            pl.BlockSpec((blk, D), lambda i: (i, 0)),
            pl.BlockSpec((NC, blk, DP), lambda i: (0, i, 0)),
            pl.BlockSpec((D, D), lambda i: (0, 0)),
            pl.BlockSpec((D, D), lambda i: (0, 0)),
            pl.BlockSpec((1, D), lambda i: (0, 0)),
            pl.BlockSpec((1, D), lambda i: (0, 0)),
        ],
        out_specs=pl.BlockSpec((blk, D), lambda i: (i, 0)),
        out_shape=jax.ShapeDtypeStruct((n, D), jnp.float32),
        name="sage_tc_combine",
    )(feat, pacc, W_self.T, W_neigh.T,
      b_self.reshape(1, D), b_neigh.reshape(1, D))
    return out


# speedup vs baseline: 1.2532x; 1.0244x over previous
import jax
import jax.numpy as jnp
from jax import lax
from jax.experimental import pallas as pl
from jax.experimental.pallas import tpu as pltpu
from jax.experimental.pallas import tpu_sc as plsc

D = 128
DP = 144
NC = 2
NS = 16
L = 16
SUB = 128


def _sc_body_maker(n_pad, units_c0, units_c1):
    stripe = n_pad // NS

    def body(feat_hbm, edges_hbm, zeros_hbm, pacc_hbm,
             idx_v, rows0, rows1, acc_sh,
             gsem0, gsem1, isem0, isem1, isem2, isem3):
        c = lax.axis_index("c")
        s = lax.axis_index("s")
        rows = (rows0, rows1)
        gsems = (gsem0, gsem1)
        isems = (isem0, isem1, isem2, isem3)

        pltpu.sync_copy(zeros_hbm, rows0.at[pl.ds(0, L)])
        zcps = [
            pltpu.async_copy(rows0.at[pl.ds(0, L)],
                             acc_sh.at[pl.ds(s * stripe + k * L, L)],
                             gsem0)
            for k in range(stripe // L)
        ]
        for cp in zcps:
            cp.wait()

        plsc.subcore_barrier()

        def pipeline(units, row_base):
            g_desc = {}
            i_desc = {}

            def idx_start(k):
                i_desc[k] = pltpu.async_copy(
                    edges_hbm.at[row_base + k], idx_v.at[k % 4],
                    isems[k % 4])

            def gather_start(k):
                g_desc[k] = pltpu.async_copy(
                    feat_hbm.at[idx_v.at[k % 4, 0]], rows[k % 2],
                    gsems[k % 2])

            def scatter_sync(k):
                pltpu.sync_copy(rows[k % 2],
                                acc_sh.at[idx_v.at[k % 4, 1]], add=True)

            idx_start(0)
            i_desc[0].wait()
            idx_start(1)
            idx_start(2)
            gather_start(0)
            for k in range(units):
                g_desc[k].wait()
                if k + 3 < units:
                    idx_start(k + 3)
                if k + 1 < units:
                    i_desc[k + 1].wait()
                    gather_start(k + 1)
                scatter_sync(k)

        @pl.when(c == 0)
        def _c0():
            pipeline(units_c0, s * units_c0)

        @pl.when(c == 1)
        def _c1():
            pipeline(units_c1, NS * units_c0 + s * units_c1)

        plsc.subcore_barrier()
        pltpu.sync_copy(acc_sh.at[pl.ds(s * stripe, stripe)],
                        pacc_hbm.at[c, pl.ds(s * stripe, stripe)])

    return body


def _tc_body(feat_ref, pacc_ref, ws_ref, wn_ref, bs_ref, bn_ref,
             out_ref):
    x = feat_ref[...]
    a = pacc_ref[0] + pacc_ref[1]
    ssum = a[:, :D]
    cnt = a[:, D:D + 1]
    h = ssum / jnp.maximum(cnt, 1.0)
    out_ref[...] = (
        jnp.dot(x, ws_ref[...], preferred_element_type=jnp.float32)
        + jnp.dot(h, wn_ref[...], preferred_element_type=jnp.float32)
        + bs_ref[...] + bn_ref[...]
    )


def kernel(feat, edge_index, W_self, b_self, W_neigh, b_neigh):
    n, d = feat.shape
    e = edge_index.shape[1]
    assert d == D

    n_pad = -(-n // (NS * L)) * (NS * L)
    tile_e = -(-e // (NC * NS * SUB)) * SUB
    e_pad = tile_e * NC * NS
    units = tile_e // SUB
    units_c0 = 112
    units_c1 = 2 * units - units_c0
    blk = 1000
    n_blocks = n // blk

    src = edge_index[0]
    dst = edge_index[1]
    pad_e = e_pad - e
    src_p = jnp.concatenate(
        [src, jnp.zeros((pad_e,), jnp.int32)]).reshape(e_pad // SUB, SUB)
    dst_p = jnp.concatenate(
        [dst, jnp.full((pad_e,), n_pad - 1, jnp.int32)]
    ).reshape(e_pad // SUB, SUB)
    edges3 = jnp.stack([src_p, dst_p], axis=1)
    featx = jnp.pad(feat, ((0, 0), (0, DP - D)), constant_values=1.0)
    zeros_blk = jnp.zeros((L, DP), jnp.float32)

    mesh = plsc.VectorSubcoreMesh(core_axis_name="c", subcore_axis_name="s")
    sc_call = pl.kernel(
        _sc_body_maker(n_pad, units_c0, units_c1),
        out_type=jax.ShapeDtypeStruct((NC, n_pad, DP), jnp.float32),
        mesh=mesh,
        scratch_types=[
            pltpu.VMEM((4, 2, SUB), jnp.int32),
            pltpu.VMEM((SUB, DP), jnp.float32),
            pltpu.VMEM((SUB, DP), jnp.float32),
            pltpu.VMEM_SHARED((n_pad, DP), jnp.float32),
        ] + [pltpu.SemaphoreType.DMA] * 6,
        compiler_params=pltpu.CompilerParams(use_tc_tiling_on_sc=False),
        name="sage_sc_aggregate",
    )
    pacc = sc_call(featx, edges3, zeros_blk)

    out = pl.pallas_call(
        _tc_body,
        grid=(n_blocks,),
        in_specs=[
            pl.BlockSpec((blk, D), lambda i: (i, 0)),
            pl.BlockSpec((NC, blk, DP), lambda i: (0, i, 0)),
            pl.BlockSpec((D, D), lambda i: (0, 0)),
            pl.BlockSpec((D, D), lambda i: (0, 0)),
            pl.BlockSpec((1, D), lambda i: (0, 0)),
            pl.BlockSpec((1, D), lambda i: (0, 0)),
        ],
        out_specs=pl.BlockSpec((blk, D), lambda i: (i, 0)),
        out_shape=jax.ShapeDtypeStruct((n, D), jnp.float32),
        name="sage_tc_combine",
    )(feat, pacc, W_self.T, W_neigh.T,
      b_self.reshape(1, D), b_neigh.reshape(1, D))
    return out
